# baseline (speedup 1.0000x reference)
import jax
import jax.numpy as jnp
from jax.experimental import pallas as pl

N = 10000
E = 320000
D_HID = 640


def _gcn_conv_max(x, W, b, src, dst, num_nodes):
    loop = jnp.arange(num_nodes, dtype=src.dtype)
    src2 = jnp.concatenate([src, loop])
    dst2 = jnp.concatenate([dst, loop])
    deg = jnp.zeros((num_nodes,), dtype=x.dtype).at[dst2].add(1.0)
    deg_inv_sqrt = jnp.where(deg > 0, jax.lax.rsqrt(jnp.maximum(deg, 1e-12)), 0.0)
    norm = deg_inv_sqrt[src2] * deg_inv_sqrt[dst2]
    h = x @ W
    msg = norm[:, None] * jnp.take(h, src2, axis=0)
    agg = jax.ops.segment_max(msg, dst2, num_segments=num_nodes)
    agg = jnp.where(jnp.isfinite(agg), agg, 0.0)
    return agg + b


def _head_kernel(h_ref, w_ref, b_ref, o_ref):
    o_ref[...] = h_ref[...] @ w_ref[...] + b_ref[...]


def kernel(x, edge_index, W1, b1, W2, b2, W3, b3):
    src = edge_index[0]
    dst = edge_index[1]
    h = _gcn_conv_max(x, W1, b1, src, dst, N)
    identity = jax.nn.relu(h)
    h = _gcn_conv_max(identity, W2, b2, src, dst, N)
    h = jax.nn.relu(h)
    h = jnp.concatenate([h, identity], axis=1)
    D_OUT = W3.shape[1]
    W3p = jnp.pad(W3, ((0, 0), (0, 128 - D_OUT)))
    b3p = jnp.pad(b3, (0, 128 - D_OUT))
    hp = jnp.pad(h, ((0, 240), (0, 0)))
    out = pl.pallas_call(
        _head_kernel,
        grid=(20,),
        in_specs=[
            pl.BlockSpec((512, 2 * D_HID), lambda i: (i, 0)),
            pl.BlockSpec((2 * D_HID, 128), lambda i: (0, 0)),
            pl.BlockSpec((128,), lambda i: (0,)),
        ],
        out_specs=pl.BlockSpec((512, 128), lambda i: (i, 0)),
        out_shape=jax.ShapeDtypeStruct((10240, 128), jnp.float32),
    )(hp, W3p, b3p)
    return out[:N, :D_OUT]


# speedup vs baseline: 2.2221x; 2.2221x over previous
import functools

import jax
import jax.numpy as jnp
from jax import lax
from jax.experimental import pallas as pl
from jax.experimental.pallas import tpu as pltpu
from jax.experimental.pallas import tpu_sc as plsc

N = 10000
NPAD = 10240
E = 320000
EB = 1024
NBLK = 313
EPAD = EB * NBLK
F = 640
NC, NS, L = 2, 16, 16
NW = NC * NS
CH = 160
CPW = NPAD // CH // NW
DCH = NPAD // NW
CAP = 48
FL = 16
R = 512

_mesh = lambda: plsc.VectorSubcoreMesh(core_axis_name="c", subcore_axis_name="s")
_sc_params = pltpu.CompilerParams(needs_layout_passes=False)


@functools.partial(
    pl.kernel,
    mesh=_mesh(),
    compiler_params=_sc_params,
    out_type=jax.ShapeDtypeStruct((NPAD,), jnp.float32),
    scratch_types=[
        pltpu.VMEM((EB,), jnp.int32),
        pltpu.VMEM((DCH,), jnp.float32),
    ],
)
def _deg_kernel(dst_hbm, deg_hbm, dstb, degl):
    wid = lax.axis_index("s") * NC + lax.axis_index("c")
    base = wid * DCH
    ones = jnp.full((L,), 1.0, jnp.float32)

    def init(i, c):
        degl[pl.ds(i * L, L)] = ones
        return c

    lax.fori_loop(0, DCH // L, init, 0)

    def blk(b, c):
        pltpu.sync_copy(dst_hbm.at[pl.ds(b * EB, EB)], dstb)

        def grp(i, c2):
            d = dstb[pl.ds(i * L, L)]
            m = (d >= base) & (d < base + DCH)
            plsc.addupdate_scatter(degl, [d - base], ones, mask=m)
            return c2

        return lax.fori_loop(0, EB // L, grp, c)

    lax.fori_loop(0, NBLK, blk, 0)
    pltpu.sync_copy(degl, deg_hbm.at[pl.ds(base, DCH)])


@functools.partial(
    pl.kernel,
    mesh=_mesh(),
    compiler_params=_sc_params,
    out_type=jax.ShapeDtypeStruct((NPAD, F), jnp.float32),
    scratch_types=[
        pltpu.VMEM((CH + 1, F), jnp.float32),
        pltpu.VMEM((FL, F), jnp.float32),
        pltpu.VMEM((EB,), jnp.int32),
        pltpu.VMEM((EB,), jnp.int32),
        pltpu.VMEM((CAP,), jnp.int32),
        pltpu.VMEM((CAP,), jnp.int32),
        pltpu.VMEM((FL,), jnp.int32),
        pltpu.SemaphoreType.DMA,
    ],
)
def _segmax_kernel(g_hbm, src_hbm, dst_hbm, out_hbm,
                   acc, rows, dstb, srcb, msrc, mdst, fidx, sem):
    wid = lax.axis_index("s") * NC + lax.axis_index("c")
    lanes = lax.iota(jnp.int32, L)
    neg1 = jnp.full((L,), -1, jnp.int32)

    def flush():
        fidx[pl.ds(0, L)] = msrc[pl.ds(0, L)]
        pltpu.async_copy(g_hbm.at[fidx], rows, sem).wait()

        def row_body(j, c):
            sl = mdst[pl.ds(0, L)]
            lj = jnp.max(jnp.where(lanes == j, sl, neg1))
            for f in range(F // L):
                a = acc[lj, pl.ds(f * L, L)]
                r = rows[j, pl.ds(f * L, L)]
                acc[lj, pl.ds(f * L, L)] = jnp.maximum(a, r)
            return c

        lax.fori_loop(0, FL, row_body, 0)

    def shift():
        for t in range(2):
            s = msrc[pl.ds(FL + t * L, L)]
            msrc[pl.ds(t * L, L)] = s
            d2 = mdst[pl.ds(FL + t * L, L)]
            mdst[pl.ds(t * L, L)] = d2

    for p in range(CPW):
        chunk = wid * CPW + p
        base = chunk * CH
        pltpu.sync_copy(g_hbm.at[pl.ds(base, CH)], acc.at[pl.ds(0, CH)])

        def blk(b, cnt):
            pltpu.sync_copy(dst_hbm.at[pl.ds(b * EB, EB)], dstb)
            pltpu.sync_copy(src_hbm.at[pl.ds(b * EB, EB)], srcb)

            def grp(i, c):
                d = dstb[pl.ds(i * L, L)]
                m = (d >= base) & (d < base + CH)
                s = srcb[pl.ds(i * L, L)]
                pos = c + jnp.cumsum(m.astype(jnp.int32)) - 1
                plsc.store_scatter(msrc, [pos], s, mask=m)
                plsc.store_scatter(mdst, [pos], d - base, mask=m)
                c = c + jnp.max(plsc.all_reduce_population_count(m))

                @pl.when(c >= FL)
                def _():
                    flush()
                    shift()

                return c - FL * (c >= FL).astype(jnp.int32)

            return lax.fori_loop(0, EB // L, grp, cnt)

        cnt = lax.fori_loop(0, NBLK, blk, 0)

        dummy_src = jnp.zeros((L,), jnp.int32)
        dummy_dst = jnp.full((L,), CH, jnp.int32)
        idxp = cnt + lanes
        mm = idxp < CAP
        plsc.store_scatter(msrc, [idxp], dummy_src, mask=mm)
        plsc.store_scatter(mdst, [idxp], dummy_dst, mask=mm)
        for _rep in range(2):
            @pl.when(cnt > 0)
            def _():
                flush()
                shift()

            cnt = jnp.maximum(cnt - FL, 0)

        pltpu.sync_copy(acc.at[pl.ds(0, CH)], out_hbm.at[pl.ds(base, CH)])


def _tc_g1_body(deg_ref, x_ref, w_ref, g_ref, dinv_ref):
    dinv = lax.rsqrt(jnp.maximum(deg_ref[...], 1e-12))
    dinv_ref[...] = dinv
    h = jnp.dot(x_ref[...], w_ref[...], preferred_element_type=jnp.float32)
    g_ref[...] = dinv * h


def _tc_mid_body(m_ref, dinv_ref, b1_ref, w2_ref, id_ref, g2_ref):
    dinv = dinv_ref[...]
    idv = jnp.maximum(dinv * m_ref[...] + b1_ref[...], 0.0)
    id_ref[...] = idv
    h2 = jnp.dot(idv, w2_ref[...], preferred_element_type=jnp.float32)
    g2_ref[...] = dinv * h2


def _tc_head_body(m_ref, dinv_ref, b2_ref, id_ref, w3a_ref, w3b_ref, b3_ref, o_ref):
    h2 = jnp.maximum(dinv_ref[...] * m_ref[...] + b2_ref[...], 0.0)
    o_ref[...] = (
        jnp.dot(h2, w3a_ref[...], preferred_element_type=jnp.float32)
        + jnp.dot(id_ref[...], w3b_ref[...], preferred_element_type=jnp.float32)
        + b3_ref[...]
    )


def kernel(x, edge_index, W1, b1, W2, b2, W3, b3):
    src = edge_index[0]
    dst = edge_index[1]
    srcp = jnp.pad(src, (0, EPAD - E))
    dstp = jnp.pad(dst, (0, EPAD - E), constant_values=-1)

    deg = _deg_kernel(dstp)
    deg2 = deg.reshape(NPAD, 1)

    xp = jnp.pad(x, ((0, NPAD - N), (0, 8 - x.shape[1])))
    W1p = jnp.pad(W1, ((0, 8 - W1.shape[0]), (0, 0)))

    g1, dinv2 = pl.pallas_call(
        _tc_g1_body,
        grid=(NPAD // R,),
        in_specs=[
            pl.BlockSpec((R, 1), lambda i: (i, 0)),
            pl.BlockSpec((R, 8), lambda i: (i, 0)),
            pl.BlockSpec((8, F), lambda i: (0, 0)),
        ],
        out_specs=[
            pl.BlockSpec((R, F), lambda i: (i, 0)),
            pl.BlockSpec((R, 1), lambda i: (i, 0)),
        ],
        out_shape=[
            jax.ShapeDtypeStruct((NPAD, F), jnp.float32),
            jax.ShapeDtypeStruct((NPAD, 1), jnp.float32),
        ],
    )(deg2, xp, W1p)

    m1 = _segmax_kernel(g1, srcp, dstp)

    identity, g2 = pl.pallas_call(
        _tc_mid_body,
        grid=(NPAD // R,),
        in_specs=[
            pl.BlockSpec((R, F), lambda i: (i, 0)),
            pl.BlockSpec((R, 1), lambda i: (i, 0)),
            pl.BlockSpec((1, F), lambda i: (0, 0)),
            pl.BlockSpec((F, F), lambda i: (0, 0)),
        ],
        out_specs=[
            pl.BlockSpec((R, F), lambda i: (i, 0)),
            pl.BlockSpec((R, F), lambda i: (i, 0)),
        ],
        out_shape=[
            jax.ShapeDtypeStruct((NPAD, F), jnp.float32),
            jax.ShapeDtypeStruct((NPAD, F), jnp.float32),
        ],
    )(m1, dinv2, b1.reshape(1, F), W2)

    m2 = _segmax_kernel(g2, srcp, dstp)

    D_OUT = W3.shape[1]
    W3p = jnp.pad(W3, ((0, 0), (0, 128 - D_OUT)))
    b3p = jnp.pad(b3, (0, 128 - D_OUT)).reshape(1, 128)

    out = pl.pallas_call(
        _tc_head_body,
        grid=(NPAD // R,),
        in_specs=[
            pl.BlockSpec((R, F), lambda i: (i, 0)),
            pl.BlockSpec((R, 1), lambda i: (i, 0)),
            pl.BlockSpec((1, F), lambda i: (0, 0)),
            pl.BlockSpec((R, F), lambda i: (i, 0)),
            pl.BlockSpec((F, 128), lambda i: (0, 0)),
            pl.BlockSpec((F, 128), lambda i: (0, 0)),
            pl.BlockSpec((1, 128), lambda i: (0, 0)),
        ],
        out_specs=pl.BlockSpec((R, 128), lambda i: (i, 0)),
        out_shape=jax.ShapeDtypeStruct((NPAD, 128), jnp.float32),
    )(m2, dinv2, b2.reshape(1, F), identity, W3p[:F], W3p[F:], b3p)

    return out[:N, :D_OUT]


# speedup vs baseline: 4.5519x; 2.0485x over previous
import functools

import jax
import jax.numpy as jnp
from jax import lax
from jax.experimental import pallas as pl
from jax.experimental.pallas import tpu as pltpu
from jax.experimental.pallas import tpu_sc as plsc

N = 10000
NPAD = 10240
E = 320000
EBA = 2048
NBLKA = 157
EPAD = EBA * NBLKA
EPADB = EPAD + 256
F = 640
NC, NS, L = 2, 16, 16
NW = NC * NS
CH = 160
NCHUNK = NPAD // CH
CPW = NCHUNK // NW
SB = 1024
R = 512
PACK = 16384

_mesh = lambda: plsc.VectorSubcoreMesh(core_axis_name="c", subcore_axis_name="s")
_sc_params = pltpu.CompilerParams(needs_layout_passes=False)


@functools.partial(
    pl.kernel,
    mesh=_mesh(),
    compiler_params=_sc_params,
    out_type=[
        jax.ShapeDtypeStruct((NPAD,), jnp.float32),
        jax.ShapeDtypeStruct((NCHUNK, EPADB), jnp.int32),
        jax.ShapeDtypeStruct((NCHUNK, L), jnp.int32),
    ],
    scratch_types=[
        pltpu.VMEM((EBA,), jnp.int32),
        pltpu.VMEM((EBA,), jnp.int32),
        pltpu.VMEM((CPW * CH,), jnp.float32),
        pltpu.VMEM((272,), jnp.int32),
        pltpu.VMEM((272,), jnp.int32),
        pltpu.VMEM((L,), jnp.int32),
    ],
)
def _bucket_kernel(dst_hbm, src_hbm, deg_hbm, bkt_hbm, cnt_hbm,
                   dstb, srcb, degl, st0, st1, crow):
    wid = lax.axis_index("s") * NC + lax.axis_index("c")
    c0id = wid * CPW
    b0 = c0id * CH
    ones = jnp.full((L,), 1.0, jnp.float32)
    zc = jnp.zeros((L,), jnp.int32)

    def init(i, c):
        degl[pl.ds(i * L, L)] = ones
        return c

    lax.fori_loop(0, CPW * CH // L, init, 0)

    def blk(b, carry):
        pltpu.sync_copy(dst_hbm.at[pl.ds(b * EBA, EBA)], dstb)
        pltpu.sync_copy(src_hbm.at[pl.ds(b * EBA, EBA)], srcb)

        def grp(i, carry2):
            c0, c1, f0, f1 = carry2
            d = dstb[pl.ds(i * L, L)]
            s = srcb[pl.ds(i * L, L)]
            ld = d - b0
            m0 = (ld >= 0) & (ld < CH)
            m1 = (ld >= CH) & (ld < 2 * CH)
            plsc.addupdate_scatter(degl, [ld], ones, mask=m0 | m1)
            pk0 = (ld * PACK) | s
            pk1 = ((ld - CH) * PACK) | s
            pos0 = c0 + jnp.cumsum(m0.astype(jnp.int32)) - 1
            plsc.store_scatter(st0, [pos0], pk0, mask=m0)
            c0 = c0 + plsc.all_reduce_population_count(m0)
            pr0 = jnp.any(c0 >= 256)

            @pl.when(pr0)
            def _():
                pltpu.sync_copy(st0.at[pl.ds(0, 256)],
                                bkt_hbm.at[c0id, pl.ds(f0 * 256, 256)])
                st0[pl.ds(0, L)] = st0[pl.ds(256, L)]

            f0 = f0 + pr0.astype(jnp.int32)
            c0 = jnp.where(c0 >= 256, c0 - 256, c0)

            pos1 = c1 + jnp.cumsum(m1.astype(jnp.int32)) - 1
            plsc.store_scatter(st1, [pos1], pk1, mask=m1)
            c1 = c1 + plsc.all_reduce_population_count(m1)
            pr1 = jnp.any(c1 >= 256)

            @pl.when(pr1)
            def _():
                pltpu.sync_copy(st1.at[pl.ds(0, 256)],
                                bkt_hbm.at[c0id + 1, pl.ds(f1 * 256, 256)])
                st1[pl.ds(0, L)] = st1[pl.ds(256, L)]

            f1 = f1 + pr1.astype(jnp.int32)
            c1 = jnp.where(c1 >= 256, c1 - 256, c1)
            return (c0, c1, f0, f1)

        return lax.fori_loop(0, EBA // L, grp, carry)

    c0, c1, f0, f1 = lax.fori_loop(0, NBLKA, blk, (zc, zc, 0, 0))
    pltpu.sync_copy(st0.at[pl.ds(0, 256)], bkt_hbm.at[c0id, pl.ds(f0 * 256, 256)])
    pltpu.sync_copy(st1.at[pl.ds(0, 256)], bkt_hbm.at[c0id + 1, pl.ds(f1 * 256, 256)])
    crow[pl.ds(0, L)] = f0 * 256 + c0
    pltpu.sync_copy(crow, cnt_hbm.at[c0id])
    crow[pl.ds(0, L)] = f1 * 256 + c1
    pltpu.sync_copy(crow, cnt_hbm.at[c0id + 1])
    pltpu.sync_copy(degl, deg_hbm.at[pl.ds(b0, CPW * CH)])


@functools.partial(
    pl.kernel,
    mesh=_mesh(),
    compiler_params=_sc_params,
    out_type=jax.ShapeDtypeStruct((NPAD, F), jnp.float32),
    scratch_types=[
        pltpu.VMEM((CH + 1, F), jnp.float32),
        pltpu.VMEM((L, F), jnp.float32),
        pltpu.VMEM((L, F), jnp.float32),
        pltpu.VMEM((SB,), jnp.int32),
        pltpu.VMEM((L,), jnp.int32),
        pltpu.VMEM((L,), jnp.int32),
        pltpu.VMEM((L,), jnp.int32),
        pltpu.SemaphoreType.DMA,
        pltpu.SemaphoreType.DMA,
    ],
)
def _segmax_kernel(g_hbm, bkt_hbm, cnt_hbm, out_hbm,
                   acc, rows0, rows1, stage, fidx0, fidx1, crow, sem0, sem1):
    wid = lax.axis_index("s") * NC + lax.axis_index("c")
    lanes = lax.iota(jnp.int32, L)
    neg1 = jnp.full((L,), -1, jnp.int32)

    def process(rows_ref, ld_vec):
        def row_body(j, c):
            lj = jnp.max(jnp.where(lanes == j, ld_vec, neg1))
            for f in range(F // L):
                a = acc[lj, pl.ds(f * L, L)]
                r = rows_ref[j, pl.ds(f * L, L)]
                acc[lj, pl.ds(f * L, L)] = jnp.maximum(a, r)
            return c

        lax.fori_loop(0, L, row_body, 0)

    for p in range(CPW):
        chunk = wid * CPW + p
        base = chunk * CH
        pltpu.sync_copy(g_hbm.at[pl.ds(base, CH)], acc.at[pl.ds(0, CH)])
        pltpu.sync_copy(cnt_hbm.at[chunk], crow)
        total = jnp.max(crow[pl.ds(0, L)])
        ngrp = (total + L - 1) // L
        pltpu.sync_copy(bkt_hbm.at[chunk, pl.ds(0, SB)], stage)
        pk = stage[pl.ds(0, L)]
        valid = lanes < total
        ld0 = jnp.where(valid, pk // PACK, CH)

        @pl.when(ngrp > 0)
        def _():
            fidx0[pl.ds(0, L)] = jnp.where(valid, pk & (PACK - 1), 0)
            pltpu.async_copy(g_hbm.at[fidx0], rows0, sem0)

        def grp_body(gi, ld_cur):
            nxt = gi + 1
            par = gi % 2

            @pl.when((nxt % (SB // L) == 0) & (nxt < ngrp))
            def _():
                pltpu.sync_copy(
                    bkt_hbm.at[chunk, pl.ds((nxt // (SB // L)) * SB, SB)], stage)

            off = nxt % (SB // L)
            pkn = stage[pl.ds(off * L, L)]
            vn = (nxt * L + lanes) < total
            sn = jnp.where(vn, pkn & (PACK - 1), 0)
            ld_nxt = jnp.where(vn, pkn // PACK, CH)

            @pl.when((par == 0) & (nxt < ngrp))
            def _():
                fidx1[pl.ds(0, L)] = sn
                pltpu.async_copy(g_hbm.at[fidx1], rows1, sem1)

            @pl.when((par == 1) & (nxt < ngrp))
            def _():
                fidx0[pl.ds(0, L)] = sn
                pltpu.async_copy(g_hbm.at[fidx0], rows0, sem0)

            @pl.when(par == 0)
            def _():
                pltpu.make_async_copy(g_hbm.at[fidx0], rows0, sem0).wait()
                process(rows0, ld_cur)

            @pl.when(par == 1)
            def _():
                pltpu.make_async_copy(g_hbm.at[fidx1], rows1, sem1).wait()
                process(rows1, ld_cur)

            return ld_nxt

        lax.fori_loop(0, ngrp, grp_body, ld0)
        pltpu.sync_copy(acc.at[pl.ds(0, CH)], out_hbm.at[pl.ds(base, CH)])


def _tc_g1_body(deg_ref, x_ref, w_ref, g_ref, dinv_ref):
    dinv = lax.rsqrt(jnp.maximum(deg_ref[...], 1e-12))
    dinv_ref[...] = dinv
    h = jnp.dot(x_ref[...], w_ref[...], preferred_element_type=jnp.float32)
    g_ref[...] = dinv * h


def _tc_mid_body(m_ref, dinv_ref, b1_ref, w2_ref, id_ref, g2_ref):
    dinv = dinv_ref[...]
    idv = jnp.maximum(dinv * m_ref[...] + b1_ref[...], 0.0)
    id_ref[...] = idv
    h2 = jnp.dot(idv, w2_ref[...], preferred_element_type=jnp.float32)
    g2_ref[...] = dinv * h2


def _tc_head_body(m_ref, dinv_ref, b2_ref, id_ref, w3a_ref, w3b_ref, b3_ref, o_ref):
    h2 = jnp.maximum(dinv_ref[...] * m_ref[...] + b2_ref[...], 0.0)
    o_ref[...] = (
        jnp.dot(h2, w3a_ref[...], preferred_element_type=jnp.float32)
        + jnp.dot(id_ref[...], w3b_ref[...], preferred_element_type=jnp.float32)
        + b3_ref[...]
    )


def kernel(x, edge_index, W1, b1, W2, b2, W3, b3):
    src = edge_index[0]
    dst = edge_index[1]
    srcp = jnp.pad(src, (0, EPAD - E))
    dstp = jnp.pad(dst, (0, EPAD - E), constant_values=-1)

    deg, bkt, cnts = _bucket_kernel(dstp, srcp)
    deg2 = deg.reshape(NPAD, 1)

    xp = jnp.pad(x, ((0, NPAD - N), (0, 8 - x.shape[1])))
    W1p = jnp.pad(W1, ((0, 8 - W1.shape[0]), (0, 0)))

    g1, dinv2 = pl.pallas_call(
        _tc_g1_body,
        grid=(NPAD // R,),
        in_specs=[
            pl.BlockSpec((R, 1), lambda i: (i, 0)),
            pl.BlockSpec((R, 8), lambda i: (i, 0)),
            pl.BlockSpec((8, F), lambda i: (0, 0)),
        ],
        out_specs=[
            pl.BlockSpec((R, F), lambda i: (i, 0)),
            pl.BlockSpec((R, 1), lambda i: (i, 0)),
        ],
        out_shape=[
            jax.ShapeDtypeStruct((NPAD, F), jnp.float32),
            jax.ShapeDtypeStruct((NPAD, 1), jnp.float32),
        ],
    )(deg2, xp, W1p)

    m1 = _segmax_kernel(g1, bkt, cnts)

    identity, g2 = pl.pallas_call(
        _tc_mid_body,
        grid=(NPAD // R,),
        in_specs=[
            pl.BlockSpec((R, F), lambda i: (i, 0)),
            pl.BlockSpec((R, 1), lambda i: (i, 0)),
            pl.BlockSpec((1, F), lambda i: (0, 0)),
            pl.BlockSpec((F, F), lambda i: (0, 0)),
        ],
        out_specs=[
            pl.BlockSpec((R, F), lambda i: (i, 0)),
            pl.BlockSpec((R, F), lambda i: (i, 0)),
        ],
        out_shape=[
            jax.ShapeDtypeStruct((NPAD, F), jnp.float32),
            jax.ShapeDtypeStruct((NPAD, F), jnp.float32),
        ],
    )(m1, dinv2, b1.reshape(1, F), W2)

    m2 = _segmax_kernel(g2, bkt, cnts)

    D_OUT = W3.shape[1]
    W3p = jnp.pad(W3, ((0, 0), (0, 128 - D_OUT)))
    b3p = jnp.pad(b3, (0, 128 - D_OUT)).reshape(1, 128)

    out = pl.pallas_call(
        _tc_head_body,
        grid=(NPAD // R,),
        in_specs=[
            pl.BlockSpec((R, F), lambda i: (i, 0)),
            pl.BlockSpec((R, 1), lambda i: (i, 0)),
            pl.BlockSpec((1, F), lambda i: (0, 0)),
            pl.BlockSpec((R, F), lambda i: (i, 0)),
            pl.BlockSpec((F, 128), lambda i: (0, 0)),
            pl.BlockSpec((F, 128), lambda i: (0, 0)),
            pl.BlockSpec((1, 128), lambda i: (0, 0)),
        ],
        out_specs=pl.BlockSpec((R, 128), lambda i: (i, 0)),
        out_shape=jax.ShapeDtypeStruct((NPAD, 128), jnp.float32),
    )(m2, dinv2, b2.reshape(1, F), identity, W3p[:F], W3p[F:], b3p)

    return out[:N, :D_OUT]
